# Initial kernel scaffold; baseline (speedup 1.0000x reference)
import jax
import jax.numpy as jnp
from jax.experimental import pallas as pl


def kernel(x, params, edge_index, batch):
    raise NotImplementedError("write your pallas kernel here")


# speedup vs baseline: 12.1950x; 12.1950x over previous
import functools

import jax
import jax.numpy as jnp
from jax import lax
from jax.experimental import pallas as pl
from jax.experimental.pallas import tpu as pltpu
from jax.experimental.pallas import tpu_sc as plsc

G = 64
NW = 32
CW = 80


def _segment_sum_sc(u, src_r, dst_r):
    n, f = u.shape
    nw, ch, cw = src_r.shape
    rows_per_tile = 8 * pl.cdiv(n, 16 * 8)
    npad = 16 * rows_per_tile
    zr = rows_per_tile // 8
    mesh = plsc.VectorSubcoreMesh(core_axis_name="c", subcore_axis_name="s")

    @functools.partial(
        pl.kernel,
        out_type=jax.ShapeDtypeStruct((2, npad, f), jnp.float32),
        mesh=mesh,
        compiler_params=pltpu.CompilerParams(use_tc_tiling_on_sc=False),
        scratch_types=[
            pltpu.VMEM((ch, cw), jnp.int32),
            pltpu.VMEM((ch, cw), jnp.int32),
            pltpu.VMEM((cw, f), jnp.float32),
            pltpu.VMEM((cw, f), jnp.float32),
            pltpu.VMEM((zr, f), jnp.float32),
            pltpu.VMEM_SHARED((npad, f), jnp.float32),
            pltpu.SemaphoreType.DMA,
            pltpu.SemaphoreType.DMA,
        ],
    )
    def seg(u_hbm, src_hbm, dst_hbm, out_hbm,
            src_v, dst_v, rows0, rows1, zbuf, acc, sem0, sem1):
        c = lax.axis_index("c")
        s = lax.axis_index("s")
        w = c * 16 + s

        pltpu.sync_copy(src_hbm.at[w], src_v)
        pltpu.sync_copy(dst_hbm.at[w], dst_v)

        @pl.loop(0, zr)
        def _(i):
            @pl.loop(0, f, step=16)
            def _(j):
                zbuf[i, pl.ds(j, 16)] = jnp.zeros((16,), jnp.float32)

        @pl.loop(0, 8)
        def _(i):
            pltpu.sync_copy(zbuf, acc.at[pl.ds(s * rows_per_tile + i * zr, zr)])

        plsc.subcore_barrier()

        rows = (rows0, rows1)
        sems = (sem0, sem1)

        def start(chunk, slot):
            pltpu.async_copy(u_hbm.at[src_v.at[chunk]], rows[slot], sems[slot])

        def wait(slot):
            pltpu.make_async_copy(u_hbm.at[src_v.at[0]], rows[slot],
                                  sems[slot]).wait()

        def scat(chunk, slot):
            pltpu.sync_copy(rows[slot], acc.at[dst_v.at[chunk]], add=True)

        start(0, 0)

        @pl.loop(0, ch, step=2)
        def _(j):
            @pl.when(j + 1 < ch)
            def _():
                start(j + 1, 1)

            wait(0)
            scat(j, 0)

            @pl.when(j + 2 < ch)
            def _():
                start(j + 2, 0)

            @pl.when(j + 1 < ch)
            def _():
                wait(1)
                scat(j + 1, 1)

        plsc.subcore_barrier()

        pltpu.sync_copy(acc.at[pl.ds(s * rows_per_tile, rows_per_tile)],
                        out_hbm.at[c].at[pl.ds(s * rows_per_tile, rows_per_tile)])

    return seg(u, src_r, dst_r)


def _first_matmul_tc(x, w1):
    n = x.shape[0]
    hd = w1.shape[1]

    def body(x_ref, w_ref, o_ref):
        o_ref[...] = jnp.dot(x_ref[...], w_ref[...],
                             preferred_element_type=jnp.float32)

    return pl.pallas_call(
        body,
        out_shape=jax.ShapeDtypeStruct((n, hd), jnp.float32),
    )(x, w1)


def _gin_tail_tc(u, parts, eps11, b1, w2, b2, gamma, beta, w1n):
    n, hd = u.shape

    def body(u_ref, p_ref, eps_ref, b1_ref, w2_ref, b2_ref,
             g_ref, be_ref, w1n_ref, h_ref, un_ref):
        agg = p_ref[0, :n, :] + p_ref[1, :n, :]
        z1 = jnp.maximum(u_ref[...] * (1.0 + eps_ref[0, 0]) + agg
                         + b1_ref[...], 0.0)
        z2 = jnp.dot(z1, w2_ref[...], preferred_element_type=jnp.float32)
        z2 = jnp.maximum(z2 + b2_ref[...], 0.0)
        mu = jnp.mean(z2, axis=0, keepdims=True)
        d = z2 - mu
        var = jnp.mean(d * d, axis=0, keepdims=True)
        h = d * lax.rsqrt(var + 1e-5) * g_ref[...] + be_ref[...]
        h_ref[...] = h
        un_ref[...] = jnp.dot(h, w1n_ref[...],
                              preferred_element_type=jnp.float32)

    return pl.pallas_call(
        body,
        out_shape=(jax.ShapeDtypeStruct((n, hd), jnp.float32),
                   jax.ShapeDtypeStruct((n, w1n.shape[1]), jnp.float32)),
    )(u, parts, eps11, b1, w2, b2, gamma, beta, w1n)


def _pool_cls_tc(h1, h2, h3, batch_row, w0a, w0b, w0c, b0, w1, b1,
                 w2, b2, w3, b3):
    n = h1.shape[0]
    c_out = w3.shape[1]

    def body(h1_ref, h2_ref, h3_ref, b_ref, w0a_ref, w0b_ref, w0c_ref,
             b0_ref, w1_ref, b1_ref, w2_ref, b2_ref, w3_ref, b3_ref, o_ref):
        gi = lax.broadcasted_iota(jnp.int32, (G, n), 0)
        pt = (gi == b_ref[...]).astype(jnp.float32)
        cnt = jnp.sum(pt, axis=1, keepdims=True)
        inv = 1.0 / jnp.maximum(cnt, 1.0)
        p1 = jnp.dot(pt, h1_ref[...], preferred_element_type=jnp.float32)
        p2 = jnp.dot(pt, h2_ref[...], preferred_element_type=jnp.float32)
        p3 = jnp.dot(pt, h3_ref[...], preferred_element_type=jnp.float32)
        g0 = (jnp.dot(p1, w0a_ref[...], preferred_element_type=jnp.float32)
              + jnp.dot(p2, w0b_ref[...], preferred_element_type=jnp.float32)
              + jnp.dot(p3, w0c_ref[...], preferred_element_type=jnp.float32))
        g0 = jnp.maximum(g0 * inv + b0_ref[...], 0.0)
        g1 = jnp.dot(g0, w1_ref[...], preferred_element_type=jnp.float32)
        g1 = jnp.maximum(g1 + b1_ref[...], 0.0)
        g2 = jnp.dot(g1, w2_ref[...], preferred_element_type=jnp.float32)
        g2 = jnp.maximum(g2 + b2_ref[...], 0.0)
        lg = jnp.dot(g2, w3_ref[...], preferred_element_type=jnp.float32)
        lg = lg + b3_ref[...]
        m = jnp.max(lg, axis=1, keepdims=True)
        ex = jnp.exp(lg - m)
        o_ref[...] = lg - m - jnp.log(jnp.sum(ex, axis=1, keepdims=True))

    return pl.pallas_call(
        body,
        out_shape=jax.ShapeDtypeStruct((G, c_out), jnp.float32),
    )(h1, h2, h3, batch_row, w0a, w0b, w0c, b0, w1, b1, w2, b2, w3, b3)


def kernel(x, params, edge_index, batch):
    n, d = x.shape
    e = edge_index.shape[1]
    ch = e // (NW * CW)
    src_r = edge_index[0].astype(jnp.int32).reshape(NW, ch, CW)
    dst_r = edge_index[1].astype(jnp.int32).reshape(NW, ch, CW)
    batch_row = batch.astype(jnp.int32).reshape(1, n)

    convs = params["convs"]
    u = _first_matmul_tc(x, convs[0]["W1"])
    hs = []
    for l, layer in enumerate(convs):
        parts = _segment_sum_sc(u, src_r, dst_r)
        w1n = convs[l + 1]["W1"] if l + 1 < len(convs) else convs[-1]["W1"]
        h, u = _gin_tail_tc(
            u, parts,
            layer["eps"].reshape(1, 1),
            layer["b1"].reshape(1, -1),
            layer["W2"], layer["b2"].reshape(1, -1),
            layer["gamma"].reshape(1, -1), layer["beta"].reshape(1, -1),
            w1n)
        hs.append(h)

    cls = params["cls"]
    hd = hs[0].shape[1]
    w0 = cls[0]["W"]
    return _pool_cls_tc(
        hs[0], hs[1], hs[2], batch_row,
        w0[:hd], w0[hd:2 * hd], w0[2 * hd:],
        cls[0]["b"].reshape(1, -1),
        cls[1]["W"], cls[1]["b"].reshape(1, -1),
        cls[2]["W"], cls[2]["b"].reshape(1, -1),
        cls[3]["W"], cls[3]["b"].reshape(1, -1))
